# baseline (speedup 1.0000x reference)
import jax
import jax.numpy as jnp
from jax import lax
from jax.experimental import pallas as pl
from jax.experimental.pallas import tpu as pltpu
from jax.experimental.pallas import tpu_sc as plsc

_NUM_CLASSES = 1000
_FEAT = 2048
_N = 8192
_SIGMA = 0.2

_NW = 32
_COLS = 64
_CG = _COLS // 16
_R = 64
_NCH = _N // _R
_NP = _NCH // 2
_TSZ = _NUM_CLASSES * _COLS
_CB = 200


def _zero_acc(acc_s, acc_c):
    z = jnp.zeros((16,), jnp.float32)

    def body(r, _):
        for g in range(_CG):
            acc_s[pl.ds(r * _COLS + g * 16, 16)] = z
        acc_c[pl.ds(r * 16, 16)] = z
        return 0
    lax.fori_loop(0, _NUM_CLASSES, body, 0)


def _chunk_start(feats, labels, j, slice0, fb, lc, sem):
    pltpu.async_copy(feats.at[pl.ds(j * _R, _R), pl.ds(slice0, 128)],
                     fb, sem)
    pltpu.async_copy(labels.at[pl.ds(j * _R, _R)], lc, sem)


def _chunk_wait(feats, labels, j, slice0, fb, lc, sem):
    pltpu.make_async_copy(feats.at[pl.ds(j * _R, _R), pl.ds(slice0, 128)],
                          fb, sem).wait()
    pltpu.make_async_copy(labels.at[pl.ds(j * _R, _R)], lc, sem).wait()


def _accumulate_chunk(acc_s, acc_c, fb, lc, coff, with_counts):
    one = jnp.ones((16,), jnp.float32)

    def rowblk(rb, _):
        lv = lc[pl.ds(rb * 16, 16)]
        cls = [lv[r] for r in range(16)]

        def load4(r4):
            return [[fb[rb * 16 + 4 * r4 + i, pl.ds(coff + g * 16, 16)]
                     for g in range(_CG)] for i in range(4)]

        def store4(r4, vals):
            for i in range(4):
                r = 4 * r4 + i
                base = cls[r] * _COLS
                for g in range(_CG):
                    plsc.addupdate(acc_s.at[pl.ds(base + g * 16, 16)],
                                   vals[i][g])
                if with_counts:
                    plsc.addupdate(acc_c.at[pl.ds(cls[r] * 16, 16)], one)

        vals = load4(0)
        for r4 in range(1, 4):
            nxt = load4(r4)
            store4(r4 - 1, vals)
            vals = nxt
        store4(3, vals)
        return 0
    lax.fori_loop(0, _R // 16, rowblk, 0)


def _accumulate_dispatch(acc_s, acc_c, fb, lc, coff, j, w):
    mine = lax.rem(j, _NW) == w

    @pl.when(mine)
    def _():
        _accumulate_chunk(acc_s, acc_c, fb, lc, coff, True)

    @pl.when(jnp.logical_not(mine))
    def _():
        _accumulate_chunk(acc_s, acc_c, fb, lc, coff, False)


def _segsum_body(rgb_f, ir_f, rgb_l, ir_l, sums, counts,
                 acc_s, acc_c, fb0, fb1, lc0, lc1, sem0, sem1):
    c = lax.axis_index("c")
    s = lax.axis_index("s")
    w = c * 16 + s
    slice0 = (w // 2) * 128
    coff = (w % 2) * 64

    for m, (feats, labels) in enumerate(((rgb_f, rgb_l), (ir_f, ir_l))):
        _zero_acc(acc_s, acc_c)
        _chunk_start(feats, labels, 0, slice0, fb0, lc0, sem0)

        def pair(p, _):
            _chunk_start(feats, labels, 2 * p + 1, slice0, fb1, lc1, sem1)
            _chunk_wait(feats, labels, 2 * p, slice0, fb0, lc0, sem0)
            _accumulate_dispatch(acc_s, acc_c, fb0, lc0, coff, 2 * p, w)

            @pl.when(p + 1 < _NP)
            def _():
                _chunk_start(feats, labels, 2 * p + 2, slice0, fb0, lc0,
                             sem0)
            _chunk_wait(feats, labels, 2 * p + 1, slice0, fb1, lc1, sem1)
            _accumulate_dispatch(acc_s, acc_c, fb1, lc1, coff, 2 * p + 1, w)
            return 0
        lax.fori_loop(0, _NP, pair, 0)

        pltpu.sync_copy(acc_s,
                        sums.at[pl.ds((m * _NW + w) * _TSZ, _TSZ)])
        pltpu.sync_copy(acc_c,
                        counts.at[pl.ds((m * _NW + w) * _NUM_CLASSES * 16,
                                        _NUM_CLASSES * 16)])


def _ema_body(vis_ref, ir_ref, sums_ref, counts_ref, out_ref):
    for m in range(2):
        cnt = jnp.sum(counts_ref[m, :, :, 0:1], axis=0)
        pres = cnt > 0.0
        factor = _SIGMA / jnp.maximum(cnt, 1.0)
        memr = vis_ref if m == 0 else ir_ref
        for w in range(_NW):
            sl = slice(w * _COLS, (w + 1) * _COLS)
            mv = memr[:, sl]
            sv = sums_ref[m, w]
            out_ref[m, :, sl] = jnp.where(
                pres, mv * (1.0 - _SIGMA) + sv * factor, mv)


@jax.jit
def _cma(rgb_feats, ir_feats, vis_memory, ir_memory, rgb_labels, ir_labels):
    mesh = plsc.VectorSubcoreMesh(core_axis_name="c", subcore_axis_name="s")
    segsum = pl.kernel(
        _segsum_body,
        out_type=(
            jax.ShapeDtypeStruct((2 * _NW * _TSZ,), jnp.float32),
            jax.ShapeDtypeStruct((2 * _NW * _NUM_CLASSES * 16,),
                                 jnp.float32),
        ),
        mesh=mesh,
        scratch_types=[
            pltpu.VMEM((_TSZ,), jnp.float32),
            pltpu.VMEM((_NUM_CLASSES * 16,), jnp.float32),
            pltpu.VMEM((_R, 128), jnp.float32),
            pltpu.VMEM((_R, 128), jnp.float32),
            pltpu.VMEM((_R,), jnp.int32),
            pltpu.VMEM((_R,), jnp.int32),
            pltpu.SemaphoreType.DMA,
            pltpu.SemaphoreType.DMA,
        ],
    )
    sums, counts = segsum(rgb_feats, ir_feats, rgb_labels, ir_labels)
    sums4 = sums.reshape(2, _NW, _NUM_CLASSES, _COLS)
    counts4 = counts.reshape(2, _NW, _NUM_CLASSES, 16)

    grid = _NUM_CLASSES // _CB
    out = pl.pallas_call(
        _ema_body,
        grid=(grid,),
        in_specs=[
            pl.BlockSpec((_CB, _FEAT), lambda g: (g, 0)),
            pl.BlockSpec((_CB, _FEAT), lambda g: (g, 0)),
            pl.BlockSpec((2, _NW, _CB, _COLS), lambda g: (0, 0, g, 0)),
            pl.BlockSpec((2, _NW, _CB, 16), lambda g: (0, 0, g, 0)),
        ],
        out_specs=pl.BlockSpec((2, _CB, _FEAT), lambda g: (0, g, 0)),
        out_shape=jax.ShapeDtypeStruct((2, _NUM_CLASSES, _FEAT),
                                       jnp.float32),
    )(vis_memory, ir_memory, sums4, counts4)
    return out


def kernel(rgb_feats, ir_feats, vis_memory, ir_memory, rgb_labels, ir_labels):
    return _cma(rgb_feats, ir_feats, vis_memory, ir_memory,
                rgb_labels.astype(jnp.int32), ir_labels.astype(jnp.int32))


# speedup vs baseline: 1.0924x; 1.0924x over previous
import jax
import jax.numpy as jnp
from jax import lax
from jax.experimental import pallas as pl
from jax.experimental.pallas import tpu as pltpu
from jax.experimental.pallas import tpu_sc as plsc

_NUM_CLASSES = 1000
_FEAT = 2048
_N = 8192
_SIGMA = 0.2

_NW = 32
_COLS = 64
_CG = _COLS // 16
_R = 64
_NCH = _N // _R
_NP = _NCH // 2
_TSZ = _NUM_CLASSES * _COLS
_CB = 200


def _zero_acc(acc_s):
    z = jnp.zeros((16,), jnp.float32)

    def body(r, _):
        for g in range(_CG):
            acc_s[pl.ds(r * _COLS + g * 16, 16)] = z
        return 0
    lax.fori_loop(0, _NUM_CLASSES, body, 0)


def _chunk_start(feats, labels, j, slice0, fb, lc, sem):
    pltpu.async_copy(feats.at[pl.ds(j * _R, _R), pl.ds(slice0, 128)],
                     fb, sem)
    pltpu.async_copy(labels.at[pl.ds(j * _R, _R)], lc, sem)


def _chunk_wait(feats, labels, j, slice0, fb, lc, sem):
    pltpu.make_async_copy(feats.at[pl.ds(j * _R, _R), pl.ds(slice0, 128)],
                          fb, sem).wait()
    pltpu.make_async_copy(labels.at[pl.ds(j * _R, _R)], lc, sem).wait()


def _accumulate_chunk(acc_s, fb, lc, coff):
    def rowblk(rb, _):
        lv = lc[pl.ds(rb * 16, 16)]
        cls = [lv[r] for r in range(16)]

        def load4(r4):
            return [[fb[rb * 16 + 4 * r4 + i, pl.ds(coff + g * 16, 16)]
                     for g in range(_CG)] for i in range(4)]

        def store4(r4, vals):
            for i in range(4):
                r = 4 * r4 + i
                base = cls[r] * _COLS
                for g in range(_CG):
                    plsc.addupdate(acc_s.at[pl.ds(base + g * 16, 16)],
                                   vals[i][g])

        vals = load4(0)
        for r4 in range(1, 4):
            nxt = load4(r4)
            store4(r4 - 1, vals)
            vals = nxt
        store4(3, vals)
        return 0
    lax.fori_loop(0, _R // 16, rowblk, 0)


def _segsum_body(rgb_f, ir_f, rgb_l, ir_l, sums,
                 acc_s, fb0, fb1, lc0, lc1, sem0, sem1):
    c = lax.axis_index("c")
    s = lax.axis_index("s")
    w = c * 16 + s
    slice0 = (w // 2) * 128
    coff = (w % 2) * 64

    for m, (feats, labels) in enumerate(((rgb_f, rgb_l), (ir_f, ir_l))):
        _zero_acc(acc_s)
        _chunk_start(feats, labels, 0, slice0, fb0, lc0, sem0)

        def pair(p, _):
            _chunk_start(feats, labels, 2 * p + 1, slice0, fb1, lc1, sem1)
            _chunk_wait(feats, labels, 2 * p, slice0, fb0, lc0, sem0)
            _accumulate_chunk(acc_s, fb0, lc0, coff)

            @pl.when(p + 1 < _NP)
            def _():
                _chunk_start(feats, labels, 2 * p + 2, slice0, fb0, lc0,
                             sem0)
            _chunk_wait(feats, labels, 2 * p + 1, slice0, fb1, lc1, sem1)
            _accumulate_chunk(acc_s, fb1, lc1, coff)
            return 0
        lax.fori_loop(0, _NP, pair, 0)

        pltpu.sync_copy(acc_s,
                        sums.at[pl.ds((m * _NW + w) * _TSZ, _TSZ)])


def _ema_body(vis_ref, ir_ref, sums_ref, rgb_l_ref, ir_l_ref, out_ref):
    cls_ids = (pl.program_id(0) * _CB
               + lax.broadcasted_iota(jnp.int32, (_CB, 1), 0))
    for m in range(2):
        lab = (rgb_l_ref if m == 0 else ir_l_ref)[:]
        eq = (lab[None, :] == cls_ids).astype(jnp.float32)
        cnt = jnp.sum(eq, axis=1, keepdims=True)
        pres = cnt > 0.0
        factor = _SIGMA / jnp.maximum(cnt, 1.0)
        memr = vis_ref if m == 0 else ir_ref
        for w in range(_NW):
            sl = slice(w * _COLS, (w + 1) * _COLS)
            mv = memr[:, sl]
            sv = sums_ref[m, w]
            out_ref[m, :, sl] = jnp.where(
                pres, mv * (1.0 - _SIGMA) + sv * factor, mv)


@jax.jit
def _cma(rgb_feats, ir_feats, vis_memory, ir_memory, rgb_labels, ir_labels):
    mesh = plsc.VectorSubcoreMesh(core_axis_name="c", subcore_axis_name="s")
    segsum = pl.kernel(
        _segsum_body,
        out_type=(
            jax.ShapeDtypeStruct((2 * _NW * _TSZ,), jnp.float32),
        ),
        mesh=mesh,
        scratch_types=[
            pltpu.VMEM((_TSZ,), jnp.float32),
            pltpu.VMEM((_R, 128), jnp.float32),
            pltpu.VMEM((_R, 128), jnp.float32),
            pltpu.VMEM((_R,), jnp.int32),
            pltpu.VMEM((_R,), jnp.int32),
            pltpu.SemaphoreType.DMA,
            pltpu.SemaphoreType.DMA,
        ],
    )
    sums, = segsum(rgb_feats, ir_feats, rgb_labels, ir_labels)
    sums4 = sums.reshape(2, _NW, _NUM_CLASSES, _COLS)

    grid = _NUM_CLASSES // _CB
    out = pl.pallas_call(
        _ema_body,
        grid=(grid,),
        in_specs=[
            pl.BlockSpec((_CB, _FEAT), lambda g: (g, 0)),
            pl.BlockSpec((_CB, _FEAT), lambda g: (g, 0)),
            pl.BlockSpec((2, _NW, _CB, _COLS), lambda g: (0, 0, g, 0)),
            pl.BlockSpec((_N,), lambda g: (0,)),
            pl.BlockSpec((_N,), lambda g: (0,)),
        ],
        out_specs=pl.BlockSpec((2, _CB, _FEAT), lambda g: (0, g, 0)),
        out_shape=jax.ShapeDtypeStruct((2, _NUM_CLASSES, _FEAT),
                                       jnp.float32),
    )(vis_memory, ir_memory, sums4, rgb_labels, ir_labels)
    return out


def kernel(rgb_feats, ir_feats, vis_memory, ir_memory, rgb_labels, ir_labels):
    return _cma(rgb_feats, ir_feats, vis_memory, ir_memory,
                rgb_labels.astype(jnp.int32), ir_labels.astype(jnp.int32))
